# Initial kernel scaffold; baseline (speedup 1.0000x reference)
import jax
import jax.numpy as jnp
from jax.experimental import pallas as pl


def kernel(x, edge_index, W11, b11, W12, b12, W13, b13, W21, b21, W22, b22, W23, b23):
    raise NotImplementedError("write your pallas kernel here")


# speedup vs baseline: 1.1375x; 1.1375x over previous
import functools

import jax
import jax.numpy as jnp
from jax import lax
from jax.experimental import pallas as pl
from jax.experimental.pallas import tpu as pltpu
from jax.experimental.pallas import tpu_sc as plsc

_f32 = jnp.float32

NTILE = 16
NCORE = 2
NW = NTILE * NCORE
K = 64
IB = 8
RB = 1000


def _mesh():
    return plsc.VectorSubcoreMesh(core_axis_name="c", subcore_axis_name="s")


@functools.lru_cache(maxsize=None)
def _sc_pass_builder(np_rows, nblk, width):
    rpt = np_rows // NTILE

    @functools.partial(
        pl.kernel,
        out_type=jax.ShapeDtypeStruct((NCORE, np_rows, width), _f32),
        mesh=_mesh(),
        scratch_types=[
            pltpu.VMEM_SHARED((np_rows, width), _f32),
            pltpu.VMEM((IB, K), jnp.int32),
            pltpu.VMEM((IB, K), jnp.int32),
            pltpu.VMEM((IB, K), jnp.int32),
            pltpu.VMEM((IB, K), jnp.int32),
            pltpu.VMEM((K, width), _f32),
            pltpu.VMEM((K, width), _f32),
            pltpu.SemaphoreType.DMA,
            pltpu.SemaphoreType.DMA,
            pltpu.SemaphoreType.DMA,
            pltpu.SemaphoreType.DMA,
        ],
    )
    def pass_kernel(y_hbm, src_hbm, dst_hbm, zeros_hbm, out_hbm,
                    acc, sb0, db0, sb1, db1, r0, r1, ms0, ms1, m0, m1):
        cid = lax.axis_index("c")
        sid = lax.axis_index("s")
        wid = cid * NTILE + sid
        pltpu.sync_copy(zeros_hbm.at[pl.ds(sid * rpt, rpt)],
                        acc.at[pl.ds(sid * rpt, rpt)])
        plsc.subcore_barrier()

        def fetch_blk(b, sb, db, ms):
            pltpu.async_copy(src_hbm.at[wid, b], sb, ms)
            pltpu.async_copy(dst_hbm.at[wid, b], db, ms)

        def wait_blk(b, sb, db, ms):
            pltpu.make_async_copy(src_hbm.at[wid, b], sb, ms).wait()
            pltpu.make_async_copy(dst_hbm.at[wid, b], db, ms).wait()

        def process_blk(sb, db):
            pltpu.async_copy(y_hbm.at[sb.at[0]], r0, m0)
            for j in range(IB):
                rc, mc = (r0, m0) if j % 2 == 0 else (r1, m1)
                rn, mn = (r1, m1) if j % 2 == 0 else (r0, m0)
                if j + 1 < IB:
                    pltpu.async_copy(y_hbm.at[sb.at[j + 1]], rn, mn)
                pltpu.make_async_copy(y_hbm.at[sb.at[j]], rc, mc).wait()
                pltpu.sync_copy(rc, acc.at[db.at[j]], add=True)

        fetch_blk(0, sb0, db0, ms0)

        def body(b2, carry):
            b = 2 * b2
            fetch_blk(b + 1, sb1, db1, ms1)
            wait_blk(b, sb0, db0, ms0)
            process_blk(sb0, db0)

            @pl.when(b + 2 < nblk)
            def _():
                fetch_blk(b + 2, sb0, db0, ms0)

            wait_blk(b + 1, sb1, db1, ms1)
            process_blk(sb1, db1)
            return carry

        lax.fori_loop(0, nblk // 2, body, 0)
        plsc.subcore_barrier()
        pltpu.sync_copy(acc.at[pl.ds(sid * rpt, rpt)],
                        out_hbm.at[cid, pl.ds(sid * rpt, rpt)])

    return pass_kernel


def _sc_pass_call(y, srcp, dstp, zeros, np_rows, nblk, width):
    return _sc_pass_builder(np_rows, nblk, width)(y, srcp, dstp, zeros)


def _deg_of(d_blk):
    return jnp.max(d_blk[0] + d_blk[1], axis=1, keepdims=True)


def _tc_layer(x, d, w1, b1, w2, b2, w3, b3, n, *, pre_norm, sum_parts):
    din = x.shape[-1]
    dout = w1.shape[1]
    dw = d.shape[-1]
    grid = n // RB
    if sum_parts:
        x_spec = pl.BlockSpec((2, RB, din), lambda i: (0, i, 0))
    else:
        x_spec = pl.BlockSpec((RB, din), lambda i: (i, 0))
    w_spec = pl.BlockSpec((din, dout), lambda i: (0, 0))
    b_spec = pl.BlockSpec((1, dout), lambda i: (0, 0))

    def body(x_ref, d_ref, w1_ref, b1_ref, w2_ref, b2_ref, w3_ref, b3_ref,
             o_ref):
        if sum_parts:
            xb = x_ref[0] + x_ref[1]
        else:
            xb = x_ref[...]
        deg = _deg_of(d_ref[...])
        nrm = lax.rsqrt(jnp.where(deg > 0.0, deg, 1.0))
        if pre_norm:
            xb = xb * nrm
        z1 = jnp.maximum(jnp.dot(xb, w1_ref[...], preferred_element_type=_f32)
                         + b1_ref[...], 0.0)
        z2 = jnp.maximum(jnp.dot(xb, w2_ref[...], preferred_element_type=_f32)
                         + b2_ref[...], 0.0)
        z3 = jnp.maximum(jnp.dot(xb, w3_ref[...], preferred_element_type=_f32)
                         + b3_ref[...], 0.0)
        o_ref[...] = (z1 + z2 + z3) * (nrm * (1.0 / 3.0))

    return pl.pallas_call(
        body,
        grid=(grid,),
        in_specs=[
            x_spec,
            pl.BlockSpec((2, RB, dw), lambda i: (0, i, 0)),
            w_spec, b_spec, w_spec, b_spec, w_spec, b_spec,
        ],
        out_specs=pl.BlockSpec((RB, dout), lambda i: (i, 0)),
        out_shape=jax.ShapeDtypeStruct((n, dout), _f32),
    )(x, d, w1, b1.reshape(1, -1), w2, b2.reshape(1, -1),
      w3, b3.reshape(1, -1))


def _tc_mid(p, d, n):
    width = p.shape[2]
    dw = d.shape[-1]
    grid = n // RB

    def body(p_ref, d_ref, o_ref):
        deg = _deg_of(d_ref[...])
        dinv = 1.0 / jnp.where(deg > 0.0, deg, 1.0)
        o_ref[...] = (p_ref[0] + p_ref[1]) * dinv

    return pl.pallas_call(
        body,
        grid=(grid,),
        in_specs=[
            pl.BlockSpec((2, RB, width), lambda i: (0, i, 0)),
            pl.BlockSpec((2, RB, dw), lambda i: (0, i, 0)),
        ],
        out_specs=pl.BlockSpec((RB, width), lambda i: (i, 0)),
        out_shape=jax.ShapeDtypeStruct((n, width), _f32),
    )(p, d)


def _tc_final(u, d, n, n_cls):
    width = u.shape[2]
    dw = d.shape[-1]
    grid = n // RB

    def body(u_ref, d_ref, o_ref):
        deg = _deg_of(d_ref[...])
        nrm = lax.rsqrt(jnp.where(deg > 0.0, deg, 1.0))
        o = (u_ref[0, :, 0:n_cls] + u_ref[1, :, 0:n_cls]) * nrm
        m = jnp.max(o, axis=1, keepdims=True)
        e = jnp.exp(o - m)
        o_ref[...] = o - m - jnp.log(jnp.sum(e, axis=1, keepdims=True))

    return pl.pallas_call(
        body,
        grid=(grid,),
        in_specs=[
            pl.BlockSpec((2, RB, width), lambda i: (0, i, 0)),
            pl.BlockSpec((2, RB, dw), lambda i: (0, i, 0)),
        ],
        out_specs=pl.BlockSpec((RB, n_cls), lambda i: (i, 0)),
        out_shape=jax.ShapeDtypeStruct((n, n_cls), _f32),
    )(u, d)


def kernel(x, edge_index, W11, b11, W12, b12, W13, b13,
           W21, b21, W22, b22, W23, b23):
    n, d_feat = x.shape
    e = edge_index.shape[1]
    n_cls = W21.shape[1]
    np_rows = (n // (NTILE * 8) + 1) * (NTILE * 8)
    nblk = -(-e // (NW * IB * K))
    nblk += nblk % 2
    ep = NW * IB * K * nblk

    src = edge_index[0].astype(jnp.int32)
    dst = edge_index[1].astype(jnp.int32)
    pad = ep - e
    srcp = jnp.concatenate(
        [src, jnp.zeros((pad,), jnp.int32)]).reshape(NW, nblk, IB, K)
    dstp = jnp.concatenate(
        [dst, jnp.full((pad,), n, jnp.int32)]).reshape(NW, nblk, IB, K)
    src_zero = jnp.zeros((NW, nblk, IB, K), jnp.int32)
    ones_mat = jnp.ones((n, d_feat), _f32)
    zh = jnp.zeros((np_rows, d_feat), _f32)
    wpad = d_feat - n_cls
    W21p = jnp.pad(W21, ((0, 0), (0, wpad)))
    W22p = jnp.pad(W22, ((0, 0), (0, wpad)))
    W23p = jnp.pad(W23, ((0, 0), (0, wpad)))
    b21p = jnp.pad(b21, (0, wpad))
    b22p = jnp.pad(b22, (0, wpad))
    b23p = jnp.pad(b23, (0, wpad))

    args = (srcp, dstp, zh, np_rows, nblk, d_feat)
    d = _sc_pass_call(ones_mat, src_zero, dstp, zh, np_rows, nblk, d_feat)

    y0 = _tc_layer(x, d, W11, b11, W12, b12, W13, b13, n,
                   pre_norm=False, sum_parts=False)
    p = _sc_pass_call(y0, *args)
    t = _tc_mid(p, d, n)
    q = _sc_pass_call(t, *args)

    y1 = _tc_layer(q, d, W21p, b21p, W22p, b22p, W23p, b23p, n,
                   pre_norm=True, sum_parts=True)
    r = _sc_pass_call(y1, *args)
    t2 = _tc_mid(r, d, n)
    u = _sc_pass_call(t2, *args)

    return _tc_final(u, d, n, n_cls)


# speedup vs baseline: 7.2227x; 6.3498x over previous
import functools

import jax
import jax.numpy as jnp
from jax import lax
from jax.experimental import pallas as pl
from jax.experimental.pallas import tpu as pltpu
from jax.experimental.pallas import tpu_sc as plsc

_f32 = jnp.float32

NTILE = 16
NCORE = 2
NW = NTILE * NCORE
K = 64
IB = 8
RB = 1000


def _mesh():
    return plsc.VectorSubcoreMesh(core_axis_name="c", subcore_axis_name="s")


@functools.lru_cache(maxsize=None)
def _sc_pass_builder(np_rows, nblk, width):
    rpt = np_rows // NTILE

    @functools.partial(
        pl.kernel,
        out_type=jax.ShapeDtypeStruct((NCORE, np_rows, width), _f32),
        mesh=_mesh(),
        scratch_types=[
            pltpu.VMEM_SHARED((np_rows, width), _f32),
            pltpu.VMEM((IB, K), jnp.int32),
            pltpu.VMEM((IB, K), jnp.int32),
            pltpu.VMEM((IB, K), jnp.int32),
            pltpu.VMEM((IB, K), jnp.int32),
            pltpu.VMEM((K, width), _f32),
            pltpu.VMEM((K, width), _f32),
            pltpu.SemaphoreType.DMA,
            pltpu.SemaphoreType.DMA,
            pltpu.SemaphoreType.DMA,
            pltpu.SemaphoreType.DMA,
        ],
    )
    def pass_kernel(y_hbm, src_hbm, dst_hbm, zeros_hbm, out_hbm,
                    acc, sb0, db0, sb1, db1, r0, r1, ms0, ms1, m0, m1):
        cid = lax.axis_index("c")
        sid = lax.axis_index("s")
        wid = cid * NTILE + sid
        pltpu.sync_copy(zeros_hbm.at[pl.ds(sid * rpt, rpt)],
                        acc.at[pl.ds(sid * rpt, rpt)])
        plsc.subcore_barrier()

        def fetch_blk(b, sb, db, ms):
            pltpu.async_copy(src_hbm.at[wid, b], sb, ms)
            pltpu.async_copy(dst_hbm.at[wid, b], db, ms)

        def wait_blk(b, sb, db, ms):
            pltpu.make_async_copy(src_hbm.at[wid, b], sb, ms).wait()
            pltpu.make_async_copy(dst_hbm.at[wid, b], db, ms).wait()

        def process_blk(sb, db):
            pltpu.async_copy(y_hbm.at[sb.at[0]], r0, m0)
            for j in range(IB):
                rc, mc = (r0, m0) if j % 2 == 0 else (r1, m1)
                rn, mn = (r1, m1) if j % 2 == 0 else (r0, m0)
                if j + 1 < IB:
                    pltpu.async_copy(y_hbm.at[sb.at[j + 1]], rn, mn)
                pltpu.make_async_copy(y_hbm.at[sb.at[j]], rc, mc).wait()
                pltpu.sync_copy(rc, acc.at[db.at[j]], add=True)

        fetch_blk(0, sb0, db0, ms0)

        def body(b2, carry):
            b = 2 * b2
            fetch_blk(b + 1, sb1, db1, ms1)
            wait_blk(b, sb0, db0, ms0)
            process_blk(sb0, db0)

            @pl.when(b + 2 < nblk)
            def _():
                fetch_blk(b + 2, sb0, db0, ms0)

            wait_blk(b + 1, sb1, db1, ms1)
            process_blk(sb1, db1)
            return carry

        lax.fori_loop(0, nblk // 2, body, 0)
        plsc.subcore_barrier()
        pltpu.sync_copy(acc.at[pl.ds(sid * rpt, rpt)],
                        out_hbm.at[cid, pl.ds(sid * rpt, rpt)])

    return pass_kernel


def _sc_pass_call(y, srcp, dstp, zeros, np_rows, nblk, width):
    return _sc_pass_builder(np_rows, nblk, width)(y, srcp, dstp, zeros)


def _deg_of(d_blk):
    return jnp.max(d_blk[0] + d_blk[1], axis=1, keepdims=True)


def _tc_layer(x, d, w1, b1, w2, b2, w3, b3, n, *, pre_norm, sum_parts):
    din = x.shape[-1]
    dout = w1.shape[1]
    dw = d.shape[-1]
    grid = n // RB
    if sum_parts:
        x_spec = pl.BlockSpec((2, RB, din), lambda i: (0, i, 0))
    else:
        x_spec = pl.BlockSpec((RB, din), lambda i: (i, 0))
    w_spec = pl.BlockSpec((din, dout), lambda i: (0, 0))
    b_spec = pl.BlockSpec((1, dout), lambda i: (0, 0))

    def body(x_ref, d_ref, w1_ref, b1_ref, w2_ref, b2_ref, w3_ref, b3_ref,
             o_ref):
        if sum_parts:
            xb = x_ref[0] + x_ref[1]
        else:
            xb = x_ref[...]
        deg = _deg_of(d_ref[...])
        nrm = lax.rsqrt(jnp.where(deg > 0.0, deg, 1.0))
        if pre_norm:
            xb = xb * nrm
        z1 = jnp.maximum(jnp.dot(xb, w1_ref[...], preferred_element_type=_f32)
                         + b1_ref[...], 0.0)
        z2 = jnp.maximum(jnp.dot(xb, w2_ref[...], preferred_element_type=_f32)
                         + b2_ref[...], 0.0)
        z3 = jnp.maximum(jnp.dot(xb, w3_ref[...], preferred_element_type=_f32)
                         + b3_ref[...], 0.0)
        o_ref[...] = (z1 + z2 + z3) * (nrm * (1.0 / 3.0))

    return pl.pallas_call(
        body,
        grid=(grid,),
        in_specs=[
            x_spec,
            pl.BlockSpec((2, RB, dw), lambda i: (0, i, 0)),
            w_spec, b_spec, w_spec, b_spec, w_spec, b_spec,
        ],
        out_specs=pl.BlockSpec((RB, dout), lambda i: (i, 0)),
        out_shape=jax.ShapeDtypeStruct((n, dout), _f32),
    )(x, d, w1, b1.reshape(1, -1), w2, b2.reshape(1, -1),
      w3, b3.reshape(1, -1))


def _tc_mid(p, d, n):
    width = p.shape[2]
    dw = d.shape[-1]
    grid = n // RB

    def body(p_ref, d_ref, o_ref):
        deg = _deg_of(d_ref[...])
        dinv = 1.0 / jnp.where(deg > 0.0, deg, 1.0)
        o_ref[...] = (p_ref[0] + p_ref[1]) * dinv

    return pl.pallas_call(
        body,
        grid=(grid,),
        in_specs=[
            pl.BlockSpec((2, RB, width), lambda i: (0, i, 0)),
            pl.BlockSpec((2, RB, dw), lambda i: (0, i, 0)),
        ],
        out_specs=pl.BlockSpec((RB, width), lambda i: (i, 0)),
        out_shape=jax.ShapeDtypeStruct((n, width), _f32),
    )(p, d)


def _tc_final(u, d, n, n_cls):
    width = u.shape[2]
    dw = d.shape[-1]
    grid = n // RB

    def body(u_ref, d_ref, o_ref):
        deg = _deg_of(d_ref[...])
        nrm = lax.rsqrt(jnp.where(deg > 0.0, deg, 1.0))
        o = (u_ref[0, :, 0:n_cls] + u_ref[1, :, 0:n_cls]) * nrm
        m = jnp.max(o, axis=1, keepdims=True)
        e = jnp.exp(o - m)
        o_ref[...] = o - m - jnp.log(jnp.sum(e, axis=1, keepdims=True))

    return pl.pallas_call(
        body,
        grid=(grid,),
        in_specs=[
            pl.BlockSpec((2, RB, width), lambda i: (0, i, 0)),
            pl.BlockSpec((2, RB, dw), lambda i: (0, i, 0)),
        ],
        out_specs=pl.BlockSpec((RB, n_cls), lambda i: (i, 0)),
        out_shape=jax.ShapeDtypeStruct((n, n_cls), _f32),
    )(u, d)


def kernel(x, edge_index, W11, b11, W12, b12, W13, b13,
           W21, b21, W22, b22, W23, b23):
    n, d_feat = x.shape
    e = edge_index.shape[1]
    n_cls = W21.shape[1]
    np_rows = (n // (NTILE * 8) + 1) * (NTILE * 8)
    nblk = -(-e // (NW * IB * K))
    nblk += nblk % 2
    ep = NW * IB * K * nblk

    src = edge_index[0].astype(jnp.int32)
    dst = edge_index[1].astype(jnp.int32)
    pad = ep - e
    srcp = jnp.concatenate(
        [src, jnp.zeros((pad,), jnp.int32)]).reshape(NW, nblk, IB, K)
    dstp = jnp.concatenate(
        [dst, jnp.full((pad,), n, jnp.int32)]).reshape(NW, nblk, IB, K)
    ones_mat = jnp.ones((n, d_feat), _f32)
    zh = jnp.zeros((np_rows, d_feat), _f32)
    wpad = d_feat - n_cls
    W21p = jnp.pad(W21, ((0, 0), (0, wpad)))
    W22p = jnp.pad(W22, ((0, 0), (0, wpad)))
    W23p = jnp.pad(W23, ((0, 0), (0, wpad)))
    b21p = jnp.pad(b21, (0, wpad))
    b22p = jnp.pad(b22, (0, wpad))
    b23p = jnp.pad(b23, (0, wpad))

    args = (srcp, dstp, zh, np_rows, nblk, d_feat)
    d = _sc_pass_call(ones_mat, srcp, dstp, zh, np_rows, nblk, d_feat)

    y0 = _tc_layer(x, d, W11, b11, W12, b12, W13, b13, n,
                   pre_norm=False, sum_parts=False)
    p = _sc_pass_call(y0, *args)
    t = _tc_mid(p, d, n)
    q = _sc_pass_call(t, *args)

    y1 = _tc_layer(q, d, W21p, b21p, W22p, b22p, W23p, b23p, n,
                   pre_norm=True, sum_parts=True)
    r = _sc_pass_call(y1, *args)
    t2 = _tc_mid(r, d, n)
    u = _sc_pass_call(t2, *args)

    return _tc_final(u, d, n, n_cls)


# speedup vs baseline: 20.2448x; 2.8029x over previous
import functools

import jax
import jax.numpy as jnp
from jax import lax
from jax.experimental import pallas as pl
from jax.experimental.pallas import tpu as pltpu
from jax.experimental.pallas import tpu_sc as plsc

_f32 = jnp.float32

NTILE = 16
NCORE = 2
NW = NTILE * NCORE
K = 64
IB = 8
RB = 1000


def _mesh():
    return plsc.VectorSubcoreMesh(core_axis_name="c", subcore_axis_name="s")


@functools.lru_cache(maxsize=None)
def _sc_pass_builder(np_rows, nblk, width):
    rpt = np_rows // NTILE

    @functools.partial(
        pl.kernel,
        out_type=jax.ShapeDtypeStruct((NCORE, np_rows, width), _f32),
        mesh=_mesh(),
        scratch_types=[
            pltpu.VMEM_SHARED((np_rows, width), _f32),
            pltpu.VMEM((IB, K), jnp.int32),
            pltpu.VMEM((IB, K), jnp.int32),
            pltpu.VMEM((IB, K), jnp.int32),
            pltpu.VMEM((IB, K), jnp.int32),
            pltpu.VMEM((K, width), _f32),
            pltpu.VMEM((K, width), _f32),
            pltpu.SemaphoreType.DMA,
            pltpu.SemaphoreType.DMA,
            pltpu.SemaphoreType.DMA,
            pltpu.SemaphoreType.DMA,
        ],
    )
    def pass_kernel(y_hbm, src_hbm, dst_hbm, zeros_hbm, out_hbm,
                    acc, sb0, db0, sb1, db1, r0, r1, ms0, ms1, m0, m1):
        cid = lax.axis_index("c")
        sid = lax.axis_index("s")
        wid = cid * NTILE + sid
        pltpu.sync_copy(zeros_hbm.at[pl.ds(sid * rpt, rpt)],
                        acc.at[pl.ds(sid * rpt, rpt)])
        plsc.subcore_barrier()

        def fetch_blk(b, sb, db, ms):
            pltpu.async_copy(src_hbm.at[wid, b], sb, ms)
            pltpu.async_copy(dst_hbm.at[wid, b], db, ms)

        def wait_blk(b, sb, db, ms):
            pltpu.make_async_copy(src_hbm.at[wid, b], sb, ms).wait()
            pltpu.make_async_copy(dst_hbm.at[wid, b], db, ms).wait()

        def process_blk(sb, db):
            pltpu.async_copy(y_hbm.at[sb.at[0]], r0, m0)
            for j in range(IB):
                rc, mc = (r0, m0) if j % 2 == 0 else (r1, m1)
                rn, mn = (r1, m1) if j % 2 == 0 else (r0, m0)
                if j + 1 < IB:
                    pltpu.async_copy(y_hbm.at[sb.at[j + 1]], rn, mn)
                pltpu.make_async_copy(y_hbm.at[sb.at[j]], rc, mc).wait()
                pltpu.sync_copy(rc, acc.at[db.at[j]], add=True)

        fetch_blk(0, sb0, db0, ms0)

        def body(b2, carry):
            b = 2 * b2
            fetch_blk(b + 1, sb1, db1, ms1)
            wait_blk(b, sb0, db0, ms0)
            process_blk(sb0, db0)

            @pl.when(b + 2 < nblk)
            def _():
                fetch_blk(b + 2, sb0, db0, ms0)

            wait_blk(b + 1, sb1, db1, ms1)
            process_blk(sb1, db1)
            return carry

        lax.fori_loop(0, nblk // 2, body, 0)
        plsc.subcore_barrier()
        pltpu.sync_copy(acc.at[pl.ds(sid * rpt, rpt)],
                        out_hbm.at[cid, pl.ds(sid * rpt, rpt)])

    return pass_kernel


def _sc_pass_call(y, srcp, dstp, zeros, np_rows, nblk, width):
    return _sc_pass_builder(np_rows, nblk, width)(y, srcp, dstp, zeros)


def _deg_of(d_blk):
    return jnp.max(d_blk[0] + d_blk[1], axis=1, keepdims=True)


def _tc_layer(x, d, w1, b1, w2, b2, w3, b3, n, *, pre_norm, sum_parts):
    din = x.shape[-1]
    dout = w1.shape[1]
    dw = d.shape[-1]
    grid = n // RB
    if sum_parts:
        x_spec = pl.BlockSpec((2, RB, din), lambda i: (0, i, 0))
    else:
        x_spec = pl.BlockSpec((RB, din), lambda i: (i, 0))
    w_spec = pl.BlockSpec((din, dout), lambda i: (0, 0))
    b_spec = pl.BlockSpec((1, dout), lambda i: (0, 0))

    def body(x_ref, d_ref, w1_ref, b1_ref, w2_ref, b2_ref, w3_ref, b3_ref,
             o_ref):
        if sum_parts:
            xb = x_ref[0] + x_ref[1]
        else:
            xb = x_ref[...]
        deg = _deg_of(d_ref[...])
        nrm = lax.rsqrt(jnp.where(deg > 0.0, deg, 1.0))
        if pre_norm:
            xb = xb * nrm
        z1 = jnp.maximum(jnp.dot(xb, w1_ref[...], preferred_element_type=_f32)
                         + b1_ref[...], 0.0)
        z2 = jnp.maximum(jnp.dot(xb, w2_ref[...], preferred_element_type=_f32)
                         + b2_ref[...], 0.0)
        z3 = jnp.maximum(jnp.dot(xb, w3_ref[...], preferred_element_type=_f32)
                         + b3_ref[...], 0.0)
        o_ref[...] = (z1 + z2 + z3) * (nrm * (1.0 / 3.0))

    return pl.pallas_call(
        body,
        grid=(grid,),
        in_specs=[
            x_spec,
            pl.BlockSpec((2, RB, dw), lambda i: (0, i, 0)),
            w_spec, b_spec, w_spec, b_spec, w_spec, b_spec,
        ],
        out_specs=pl.BlockSpec((RB, dout), lambda i: (i, 0)),
        out_shape=jax.ShapeDtypeStruct((n, dout), _f32),
    )(x, d, w1, b1.reshape(1, -1), w2, b2.reshape(1, -1),
      w3, b3.reshape(1, -1))


def _tc_mid(p, d, n):
    width = p.shape[2]
    dw = d.shape[-1]
    grid = n // RB

    def body(p_ref, d_ref, o_ref):
        deg = _deg_of(d_ref[...])
        dinv = 1.0 / jnp.where(deg > 0.0, deg, 1.0)
        o_ref[...] = (p_ref[0] + p_ref[1]) * dinv

    return pl.pallas_call(
        body,
        grid=(grid,),
        in_specs=[
            pl.BlockSpec((2, RB, width), lambda i: (0, i, 0)),
            pl.BlockSpec((2, RB, dw), lambda i: (0, i, 0)),
        ],
        out_specs=pl.BlockSpec((RB, width), lambda i: (i, 0)),
        out_shape=jax.ShapeDtypeStruct((n, width), _f32),
    )(p, d)


def _tc_final(u, d, n, n_cls):
    width = u.shape[2]
    dw = d.shape[-1]
    grid = n // RB

    def body(u_ref, d_ref, o_ref):
        deg = _deg_of(d_ref[...])
        nrm = lax.rsqrt(jnp.where(deg > 0.0, deg, 1.0))
        o = (u_ref[0, :, 0:n_cls] + u_ref[1, :, 0:n_cls]) * nrm
        m = jnp.max(o, axis=1, keepdims=True)
        e = jnp.exp(o - m)
        o_ref[...] = o - m - jnp.log(jnp.sum(e, axis=1, keepdims=True))

    return pl.pallas_call(
        body,
        grid=(grid,),
        in_specs=[
            pl.BlockSpec((2, RB, width), lambda i: (0, i, 0)),
            pl.BlockSpec((2, RB, dw), lambda i: (0, i, 0)),
        ],
        out_specs=pl.BlockSpec((RB, n_cls), lambda i: (i, 0)),
        out_shape=jax.ShapeDtypeStruct((n, n_cls), _f32),
    )(u, d)


def kernel(x, edge_index, W11, b11, W12, b12, W13, b13,
           W21, b21, W22, b22, W23, b23):
    n, d_feat = x.shape
    e = edge_index.shape[1]
    n_cls = W21.shape[1]
    np_rows = (n // (NTILE * 8) + 1) * (NTILE * 8)
    nblk = -(-e // (NW * IB * K))
    nblk += nblk % 2
    ep = NW * IB * K * nblk

    src = edge_index[0].astype(jnp.int32)
    dst = edge_index[1].astype(jnp.int32)
    pad = ep - e
    pad_ids = jnp.arange(pad, dtype=jnp.int32)
    srcp = jnp.concatenate(
        [src, pad_ids % jnp.int32(n)]).reshape(NW, nblk, IB, K)
    dstp = jnp.concatenate(
        [dst, jnp.int32(n) + pad_ids % jnp.int32(np_rows - n)]
    ).reshape(NW, nblk, IB, K)
    ones_mat = jnp.ones((n, d_feat), _f32)
    zh = jnp.zeros((np_rows, d_feat), _f32)
    wpad = d_feat - n_cls
    W21p = jnp.pad(W21, ((0, 0), (0, wpad)))
    W22p = jnp.pad(W22, ((0, 0), (0, wpad)))
    W23p = jnp.pad(W23, ((0, 0), (0, wpad)))
    b21p = jnp.pad(b21, (0, wpad))
    b22p = jnp.pad(b22, (0, wpad))
    b23p = jnp.pad(b23, (0, wpad))

    args = (srcp, dstp, zh, np_rows, nblk, d_feat)
    d = _sc_pass_call(ones_mat, srcp, dstp, zh, np_rows, nblk, d_feat)

    y0 = _tc_layer(x, d, W11, b11, W12, b12, W13, b13, n,
                   pre_norm=False, sum_parts=False)
    p = _sc_pass_call(y0, *args)
    t = _tc_mid(p, d, n)
    q = _sc_pass_call(t, *args)

    y1 = _tc_layer(q, d, W21p, b21p, W22p, b22p, W23p, b23p, n,
                   pre_norm=True, sum_parts=True)
    r = _sc_pass_call(y1, *args)
    t2 = _tc_mid(r, d, n)
    u = _sc_pass_call(t2, *args)

    return _tc_final(u, d, n, n_cls)


# speedup vs baseline: 24.0792x; 1.1894x over previous
import functools

import jax
import jax.numpy as jnp
from jax import lax
from jax.experimental import pallas as pl
from jax.experimental.pallas import tpu as pltpu
from jax.experimental.pallas import tpu_sc as plsc

_f32 = jnp.float32

NTILE = 16
NCORE = 2
NW = NTILE * NCORE
K = 64
IB = 8
RB = 1000


def _mesh():
    return plsc.VectorSubcoreMesh(core_axis_name="c", subcore_axis_name="s")


@functools.lru_cache(maxsize=None)
def _sc_pass_builder(np_rows, nblk, width):
    rpt = np_rows // NTILE

    @functools.partial(
        pl.kernel,
        out_type=jax.ShapeDtypeStruct((NCORE, np_rows, width), _f32),
        mesh=_mesh(),
        scratch_types=[
            pltpu.VMEM_SHARED((np_rows, width), _f32),
            pltpu.VMEM((IB, K), jnp.int32),
            pltpu.VMEM((IB, K), jnp.int32),
            pltpu.VMEM((IB, K), jnp.int32),
            pltpu.VMEM((IB, K), jnp.int32),
            pltpu.VMEM((K, width), _f32),
            pltpu.VMEM((K, width), _f32),
            pltpu.VMEM((K, width), _f32),
            pltpu.VMEM((K, width), _f32),
            pltpu.SemaphoreType.DMA,
            pltpu.SemaphoreType.DMA,
            pltpu.SemaphoreType.DMA,
            pltpu.SemaphoreType.DMA,
            pltpu.SemaphoreType.DMA,
            pltpu.SemaphoreType.DMA,
        ],
    )
    def pass_kernel(y_hbm, src_hbm, dst_hbm, zeros_hbm, out_hbm,
                    acc, sb0, db0, sb1, db1, r0, r1, r2, r3,
                    ms0, ms1, m0, m1, m2, m3):
        cid = lax.axis_index("c")
        sid = lax.axis_index("s")
        wid = cid * NTILE + sid
        pltpu.sync_copy(zeros_hbm.at[pl.ds(sid * rpt, rpt)],
                        acc.at[pl.ds(sid * rpt, rpt)])
        plsc.subcore_barrier()

        def fetch_blk(b, sb, db, ms):
            pltpu.async_copy(src_hbm.at[wid, b], sb, ms)
            pltpu.async_copy(dst_hbm.at[wid, b], db, ms)

        def wait_blk(b, sb, db, ms):
            pltpu.make_async_copy(src_hbm.at[wid, b], sb, ms).wait()
            pltpu.make_async_copy(dst_hbm.at[wid, b], db, ms).wait()

        bufs = ((r0, m0), (r1, m1), (r2, m2), (r3, m3))
        depth = 3

        def process_blk(sb, db):
            for j in range(min(depth, IB)):
                rb, mb = bufs[j % 4]
                pltpu.async_copy(y_hbm.at[sb.at[j]], rb, mb)
            for j in range(IB):
                rc, mc = bufs[j % 4]
                pltpu.make_async_copy(y_hbm.at[sb.at[j]], rc, mc).wait()
                if j + depth < IB:
                    rn, mn = bufs[(j + depth) % 4]
                    pltpu.async_copy(y_hbm.at[sb.at[j + depth]], rn, mn)
                pltpu.sync_copy(rc, acc.at[db.at[j]], add=True)

        fetch_blk(0, sb0, db0, ms0)

        def body(b2, carry):
            b = 2 * b2
            fetch_blk(b + 1, sb1, db1, ms1)
            wait_blk(b, sb0, db0, ms0)
            process_blk(sb0, db0)

            @pl.when(b + 2 < nblk)
            def _():
                fetch_blk(b + 2, sb0, db0, ms0)

            wait_blk(b + 1, sb1, db1, ms1)
            process_blk(sb1, db1)
            return carry

        lax.fori_loop(0, nblk // 2, body, 0)
        plsc.subcore_barrier()
        pltpu.sync_copy(acc.at[pl.ds(sid * rpt, rpt)],
                        out_hbm.at[cid, pl.ds(sid * rpt, rpt)])

    return pass_kernel


def _sc_pass_call(y, srcp, dstp, zeros, np_rows, nblk, width):
    return _sc_pass_builder(np_rows, nblk, width)(y, srcp, dstp, zeros)


def _deg_of(d_blk):
    return jnp.max(d_blk[0] + d_blk[1], axis=1, keepdims=True)


def _tc_layer(x, d, w1, b1, w2, b2, w3, b3, n, *, pre_norm, sum_parts):
    din = x.shape[-1]
    dout = w1.shape[1]
    dw = d.shape[-1]
    grid = n // RB
    if sum_parts:
        x_spec = pl.BlockSpec((2, RB, din), lambda i: (0, i, 0))
    else:
        x_spec = pl.BlockSpec((RB, din), lambda i: (i, 0))
    w_spec = pl.BlockSpec((din, dout), lambda i: (0, 0))
    b_spec = pl.BlockSpec((1, dout), lambda i: (0, 0))

    def body(x_ref, d_ref, w1_ref, b1_ref, w2_ref, b2_ref, w3_ref, b3_ref,
             o_ref):
        if sum_parts:
            xb = x_ref[0] + x_ref[1]
        else:
            xb = x_ref[...]
        deg = _deg_of(d_ref[...])
        nrm = lax.rsqrt(jnp.where(deg > 0.0, deg, 1.0))
        if pre_norm:
            xb = xb * nrm
        z1 = jnp.maximum(jnp.dot(xb, w1_ref[...], preferred_element_type=_f32)
                         + b1_ref[...], 0.0)
        z2 = jnp.maximum(jnp.dot(xb, w2_ref[...], preferred_element_type=_f32)
                         + b2_ref[...], 0.0)
        z3 = jnp.maximum(jnp.dot(xb, w3_ref[...], preferred_element_type=_f32)
                         + b3_ref[...], 0.0)
        o_ref[...] = (z1 + z2 + z3) * (nrm * (1.0 / 3.0))

    return pl.pallas_call(
        body,
        grid=(grid,),
        in_specs=[
            x_spec,
            pl.BlockSpec((2, RB, dw), lambda i: (0, i, 0)),
            w_spec, b_spec, w_spec, b_spec, w_spec, b_spec,
        ],
        out_specs=pl.BlockSpec((RB, dout), lambda i: (i, 0)),
        out_shape=jax.ShapeDtypeStruct((n, dout), _f32),
    )(x, d, w1, b1.reshape(1, -1), w2, b2.reshape(1, -1),
      w3, b3.reshape(1, -1))


def _tc_mid(p, d, n):
    width = p.shape[2]
    dw = d.shape[-1]
    grid = n // RB

    def body(p_ref, d_ref, o_ref):
        deg = _deg_of(d_ref[...])
        dinv = 1.0 / jnp.where(deg > 0.0, deg, 1.0)
        o_ref[...] = (p_ref[0] + p_ref[1]) * dinv

    return pl.pallas_call(
        body,
        grid=(grid,),
        in_specs=[
            pl.BlockSpec((2, RB, width), lambda i: (0, i, 0)),
            pl.BlockSpec((2, RB, dw), lambda i: (0, i, 0)),
        ],
        out_specs=pl.BlockSpec((RB, width), lambda i: (i, 0)),
        out_shape=jax.ShapeDtypeStruct((n, width), _f32),
    )(p, d)


def _tc_final(u, d, n, n_cls):
    width = u.shape[2]
    dw = d.shape[-1]
    grid = n // RB

    def body(u_ref, d_ref, o_ref):
        deg = _deg_of(d_ref[...])
        nrm = lax.rsqrt(jnp.where(deg > 0.0, deg, 1.0))
        o = (u_ref[0, :, 0:n_cls] + u_ref[1, :, 0:n_cls]) * nrm
        m = jnp.max(o, axis=1, keepdims=True)
        e = jnp.exp(o - m)
        o_ref[...] = o - m - jnp.log(jnp.sum(e, axis=1, keepdims=True))

    return pl.pallas_call(
        body,
        grid=(grid,),
        in_specs=[
            pl.BlockSpec((2, RB, width), lambda i: (0, i, 0)),
            pl.BlockSpec((2, RB, dw), lambda i: (0, i, 0)),
        ],
        out_specs=pl.BlockSpec((RB, n_cls), lambda i: (i, 0)),
        out_shape=jax.ShapeDtypeStruct((n, n_cls), _f32),
    )(u, d)


def kernel(x, edge_index, W11, b11, W12, b12, W13, b13,
           W21, b21, W22, b22, W23, b23):
    n, d_feat = x.shape
    e = edge_index.shape[1]
    n_cls = W21.shape[1]
    np_rows = (n // (NTILE * 8) + 1) * (NTILE * 8)
    nblk = -(-e // (NW * IB * K))
    nblk += nblk % 2
    ep = NW * IB * K * nblk

    src = edge_index[0].astype(jnp.int32)
    dst = edge_index[1].astype(jnp.int32)
    pad = ep - e
    pad_ids = jnp.arange(pad, dtype=jnp.int32)
    srcp = jnp.concatenate(
        [src, pad_ids % jnp.int32(n)]).reshape(NW, nblk, IB, K)
    dstp = jnp.concatenate(
        [dst, jnp.int32(n) + pad_ids % jnp.int32(np_rows - n)]
    ).reshape(NW, nblk, IB, K)
    ones_mat = jnp.ones((n, d_feat), _f32)
    zh = jnp.zeros((np_rows, d_feat), _f32)
    wpad = d_feat - n_cls
    W21p = jnp.pad(W21, ((0, 0), (0, wpad)))
    W22p = jnp.pad(W22, ((0, 0), (0, wpad)))
    W23p = jnp.pad(W23, ((0, 0), (0, wpad)))
    b21p = jnp.pad(b21, (0, wpad))
    b22p = jnp.pad(b22, (0, wpad))
    b23p = jnp.pad(b23, (0, wpad))

    args = (srcp, dstp, zh, np_rows, nblk, d_feat)
    d = _sc_pass_call(ones_mat, srcp, dstp, zh, np_rows, nblk, d_feat)

    y0 = _tc_layer(x, d, W11, b11, W12, b12, W13, b13, n,
                   pre_norm=False, sum_parts=False)
    p = _sc_pass_call(y0, *args)
    t = _tc_mid(p, d, n)
    q = _sc_pass_call(t, *args)

    y1 = _tc_layer(q, d, W21p, b21p, W22p, b22p, W23p, b23p, n,
                   pre_norm=True, sum_parts=True)
    r = _sc_pass_call(y1, *args)
    t2 = _tc_mid(r, d, n)
    u = _sc_pass_call(t2, *args)

    return _tc_final(u, d, n, n_cls)


# speedup vs baseline: 25.5538x; 1.0612x over previous
import functools

import jax
import jax.numpy as jnp
from jax import lax
from jax.experimental import pallas as pl
from jax.experimental.pallas import tpu as pltpu
from jax.experimental.pallas import tpu_sc as plsc

_f32 = jnp.float32

NTILE = 16
NCORE = 2
NW = NTILE * NCORE
K = 64
IB = 16
RB = 1000


def _mesh():
    return plsc.VectorSubcoreMesh(core_axis_name="c", subcore_axis_name="s")


@functools.lru_cache(maxsize=None)
def _sc_pass_builder(np_rows, nblk, width):
    rpt = np_rows // NTILE

    @functools.partial(
        pl.kernel,
        out_type=jax.ShapeDtypeStruct((NCORE, np_rows, width), _f32),
        mesh=_mesh(),
        scratch_types=[
            pltpu.VMEM_SHARED((np_rows, width), _f32),
            pltpu.VMEM((IB, K), jnp.int32),
            pltpu.VMEM((IB, K), jnp.int32),
            pltpu.VMEM((IB, K), jnp.int32),
            pltpu.VMEM((IB, K), jnp.int32),
            pltpu.VMEM((K, width), _f32),
            pltpu.VMEM((K, width), _f32),
            pltpu.VMEM((K, width), _f32),
            pltpu.VMEM((K, width), _f32),
            pltpu.VMEM((K, width), _f32),
            pltpu.SemaphoreType.DMA,
            pltpu.SemaphoreType.DMA,
            pltpu.SemaphoreType.DMA,
            pltpu.SemaphoreType.DMA,
            pltpu.SemaphoreType.DMA,
            pltpu.SemaphoreType.DMA,
            pltpu.SemaphoreType.DMA,
        ],
    )
    def pass_kernel(y_hbm, src_hbm, dst_hbm, zeros_hbm, out_hbm,
                    acc, sb0, db0, sb1, db1, r0, r1, r2, r3, r4,
                    ms0, ms1, m0, m1, m2, m3, m4):
        cid = lax.axis_index("c")
        sid = lax.axis_index("s")
        wid = cid * NTILE + sid
        pltpu.sync_copy(zeros_hbm.at[pl.ds(sid * rpt, rpt)],
                        acc.at[pl.ds(sid * rpt, rpt)])
        plsc.subcore_barrier()

        def fetch_blk(b, sb, db, ms):
            pltpu.async_copy(src_hbm.at[wid, b], sb, ms)
            pltpu.async_copy(dst_hbm.at[wid, b], db, ms)

        def wait_blk(b, sb, db, ms):
            pltpu.make_async_copy(src_hbm.at[wid, b], sb, ms).wait()
            pltpu.make_async_copy(dst_hbm.at[wid, b], db, ms).wait()

        bufs = ((r0, m0), (r1, m1), (r2, m2), (r3, m3), (r4, m4))
        nbuf = len(bufs)
        depth = 4

        def process_blk(sb, db):
            for j in range(min(depth, IB)):
                rb, mb = bufs[j % nbuf]
                pltpu.async_copy(y_hbm.at[sb.at[j]], rb, mb)
            for j in range(IB):
                rc, mc = bufs[j % nbuf]
                pltpu.make_async_copy(y_hbm.at[sb.at[j]], rc, mc).wait()
                if j + depth < IB:
                    rn, mn = bufs[(j + depth) % nbuf]
                    pltpu.async_copy(y_hbm.at[sb.at[j + depth]], rn, mn)
                pltpu.sync_copy(rc, acc.at[db.at[j]], add=True)

        fetch_blk(0, sb0, db0, ms0)

        def body(b2, carry):
            b = 2 * b2
            fetch_blk(b + 1, sb1, db1, ms1)
            wait_blk(b, sb0, db0, ms0)
            process_blk(sb0, db0)

            @pl.when(b + 2 < nblk)
            def _():
                fetch_blk(b + 2, sb0, db0, ms0)

            wait_blk(b + 1, sb1, db1, ms1)
            process_blk(sb1, db1)
            return carry

        lax.fori_loop(0, nblk // 2, body, 0)
        plsc.subcore_barrier()
        pltpu.sync_copy(acc.at[pl.ds(sid * rpt, rpt)],
                        out_hbm.at[cid, pl.ds(sid * rpt, rpt)])

    return pass_kernel


def _sc_pass_call(y, srcp, dstp, zeros, np_rows, nblk, width):
    return _sc_pass_builder(np_rows, nblk, width)(y, srcp, dstp, zeros)


def _deg_of(d_blk):
    return jnp.max(d_blk[0] + d_blk[1], axis=1, keepdims=True)


def _tc_layer(x, d, w1, b1, w2, b2, w3, b3, n, *, pre_norm, sum_parts):
    din = x.shape[-1]
    dout = w1.shape[1]
    dw = d.shape[-1]
    grid = n // RB
    if sum_parts:
        x_spec = pl.BlockSpec((2, RB, din), lambda i: (0, i, 0))
    else:
        x_spec = pl.BlockSpec((RB, din), lambda i: (i, 0))
    w_spec = pl.BlockSpec((din, dout), lambda i: (0, 0))
    b_spec = pl.BlockSpec((1, dout), lambda i: (0, 0))

    def body(x_ref, d_ref, w1_ref, b1_ref, w2_ref, b2_ref, w3_ref, b3_ref,
             o_ref):
        if sum_parts:
            xb = x_ref[0] + x_ref[1]
        else:
            xb = x_ref[...]
        deg = _deg_of(d_ref[...])
        nrm = lax.rsqrt(jnp.where(deg > 0.0, deg, 1.0))
        if pre_norm:
            xb = xb * nrm
        z1 = jnp.maximum(jnp.dot(xb, w1_ref[...], preferred_element_type=_f32)
                         + b1_ref[...], 0.0)
        z2 = jnp.maximum(jnp.dot(xb, w2_ref[...], preferred_element_type=_f32)
                         + b2_ref[...], 0.0)
        z3 = jnp.maximum(jnp.dot(xb, w3_ref[...], preferred_element_type=_f32)
                         + b3_ref[...], 0.0)
        o_ref[...] = (z1 + z2 + z3) * (nrm * (1.0 / 3.0))

    return pl.pallas_call(
        body,
        grid=(grid,),
        in_specs=[
            x_spec,
            pl.BlockSpec((2, RB, dw), lambda i: (0, i, 0)),
            w_spec, b_spec, w_spec, b_spec, w_spec, b_spec,
        ],
        out_specs=pl.BlockSpec((RB, dout), lambda i: (i, 0)),
        out_shape=jax.ShapeDtypeStruct((n, dout), _f32),
    )(x, d, w1, b1.reshape(1, -1), w2, b2.reshape(1, -1),
      w3, b3.reshape(1, -1))


def _tc_mid(p, d, n):
    width = p.shape[2]
    dw = d.shape[-1]
    grid = n // RB

    def body(p_ref, d_ref, o_ref):
        deg = _deg_of(d_ref[...])
        dinv = 1.0 / jnp.where(deg > 0.0, deg, 1.0)
        o_ref[...] = (p_ref[0] + p_ref[1]) * dinv

    return pl.pallas_call(
        body,
        grid=(grid,),
        in_specs=[
            pl.BlockSpec((2, RB, width), lambda i: (0, i, 0)),
            pl.BlockSpec((2, RB, dw), lambda i: (0, i, 0)),
        ],
        out_specs=pl.BlockSpec((RB, width), lambda i: (i, 0)),
        out_shape=jax.ShapeDtypeStruct((n, width), _f32),
    )(p, d)


def _tc_final(u, d, n, n_cls):
    width = u.shape[2]
    dw = d.shape[-1]
    grid = n // RB

    def body(u_ref, d_ref, o_ref):
        deg = _deg_of(d_ref[...])
        nrm = lax.rsqrt(jnp.where(deg > 0.0, deg, 1.0))
        o = (u_ref[0, :, 0:n_cls] + u_ref[1, :, 0:n_cls]) * nrm
        m = jnp.max(o, axis=1, keepdims=True)
        e = jnp.exp(o - m)
        o_ref[...] = o - m - jnp.log(jnp.sum(e, axis=1, keepdims=True))

    return pl.pallas_call(
        body,
        grid=(grid,),
        in_specs=[
            pl.BlockSpec((2, RB, width), lambda i: (0, i, 0)),
            pl.BlockSpec((2, RB, dw), lambda i: (0, i, 0)),
        ],
        out_specs=pl.BlockSpec((RB, n_cls), lambda i: (i, 0)),
        out_shape=jax.ShapeDtypeStruct((n, n_cls), _f32),
    )(u, d)


def kernel(x, edge_index, W11, b11, W12, b12, W13, b13,
           W21, b21, W22, b22, W23, b23):
    n, d_feat = x.shape
    e = edge_index.shape[1]
    n_cls = W21.shape[1]
    np_rows = (n // (NTILE * 8) + 1) * (NTILE * 8)
    nblk = -(-e // (NW * IB * K))
    nblk += nblk % 2
    ep = NW * IB * K * nblk

    src = edge_index[0].astype(jnp.int32)
    dst = edge_index[1].astype(jnp.int32)
    pad = ep - e
    pad_ids = jnp.arange(pad, dtype=jnp.int32)
    srcp = jnp.concatenate(
        [src, pad_ids % jnp.int32(n)]).reshape(NW, nblk, IB, K)
    dstp = jnp.concatenate(
        [dst, jnp.int32(n) + pad_ids % jnp.int32(np_rows - n)]
    ).reshape(NW, nblk, IB, K)
    ones_mat = jnp.ones((n, d_feat), _f32)
    zh = jnp.zeros((np_rows, d_feat), _f32)
    wpad = d_feat - n_cls
    W21p = jnp.pad(W21, ((0, 0), (0, wpad)))
    W22p = jnp.pad(W22, ((0, 0), (0, wpad)))
    W23p = jnp.pad(W23, ((0, 0), (0, wpad)))
    b21p = jnp.pad(b21, (0, wpad))
    b22p = jnp.pad(b22, (0, wpad))
    b23p = jnp.pad(b23, (0, wpad))

    args = (srcp, dstp, zh, np_rows, nblk, d_feat)
    d = _sc_pass_call(ones_mat, srcp, dstp, zh, np_rows, nblk, d_feat)

    y0 = _tc_layer(x, d, W11, b11, W12, b12, W13, b13, n,
                   pre_norm=False, sum_parts=False)
    p = _sc_pass_call(y0, *args)
    t = _tc_mid(p, d, n)
    q = _sc_pass_call(t, *args)

    y1 = _tc_layer(q, d, W21p, b21p, W22p, b22p, W23p, b23p, n,
                   pre_norm=True, sum_parts=True)
    r = _sc_pass_call(y1, *args)
    t2 = _tc_mid(r, d, n)
    u = _sc_pass_call(t2, *args)

    return _tc_final(u, d, n, n_cls)


# speedup vs baseline: 25.9393x; 1.0151x over previous
import functools

import jax
import jax.numpy as jnp
from jax import lax
from jax.experimental import pallas as pl
from jax.experimental.pallas import tpu as pltpu
from jax.experimental.pallas import tpu_sc as plsc

_f32 = jnp.float32

NTILE = 16
NCORE = 2
NW = NTILE * NCORE
K = 64
IB = 16
RB = 1000


def _mesh():
    return plsc.VectorSubcoreMesh(core_axis_name="c", subcore_axis_name="s")


@functools.lru_cache(maxsize=None)
def _sc_pass_builder(np_rows, nblk, width):
    rpt = np_rows // NTILE

    @functools.partial(
        pl.kernel,
        out_type=jax.ShapeDtypeStruct((NCORE, np_rows, width), _f32),
        mesh=_mesh(),
        scratch_types=[
            pltpu.VMEM_SHARED((np_rows, width), _f32),
            pltpu.VMEM((IB, K), jnp.int32),
            pltpu.VMEM((IB, K), jnp.int32),
            pltpu.VMEM((IB, K), jnp.int32),
            pltpu.VMEM((IB, K), jnp.int32),
            pltpu.VMEM((K, width), _f32),
            pltpu.VMEM((K, width), _f32),
            pltpu.VMEM((K, width), _f32),
            pltpu.VMEM((K, width), _f32),
            pltpu.VMEM((K, width), _f32),
            pltpu.SemaphoreType.DMA,
            pltpu.SemaphoreType.DMA,
            pltpu.SemaphoreType.DMA,
            pltpu.SemaphoreType.DMA,
            pltpu.SemaphoreType.DMA,
            pltpu.SemaphoreType.DMA,
            pltpu.SemaphoreType.DMA,
            pltpu.SemaphoreType.DMA,
            pltpu.SemaphoreType.DMA,
            pltpu.SemaphoreType.DMA,
            pltpu.SemaphoreType.DMA,
            pltpu.SemaphoreType.DMA,
        ],
    )
    def pass_kernel(y_hbm, src_hbm, dst_hbm, zeros_hbm, out_hbm,
                    acc, sb0, db0, sb1, db1, r0, r1, r2, r3, r4,
                    ms0, ms1, m0, m1, m2, m3, m4, c0, c1, c2, c3, c4):
        cid = lax.axis_index("c")
        sid = lax.axis_index("s")
        wid = cid * NTILE + sid
        pltpu.sync_copy(zeros_hbm.at[pl.ds(sid * rpt, rpt)],
                        acc.at[pl.ds(sid * rpt, rpt)])
        plsc.subcore_barrier()

        def fetch_blk(b, sb, db, ms):
            pltpu.async_copy(src_hbm.at[wid, b], sb, ms)
            pltpu.async_copy(dst_hbm.at[wid, b], db, ms)

        def wait_blk(b, sb, db, ms):
            pltpu.make_async_copy(src_hbm.at[wid, b], sb, ms).wait()
            pltpu.make_async_copy(dst_hbm.at[wid, b], db, ms).wait()

        bufs = ((r0, m0, c0), (r1, m1, c1), (r2, m2, c2), (r3, m3, c3),
                (r4, m4, c4))
        nbuf = len(bufs)
        depth = 4

        def wait_scatter(bufi, db):
            rb, _, cs = bufs[bufi]
            pltpu.make_async_copy(rb, acc.at[db.at[0]], cs).wait()

        def process_blk(sb, db, not_first):
            def guarded_wait(bufi):
                if not_first is None:
                    wait_scatter(bufi, db)
                else:
                    @pl.when(not_first)
                    def _():
                        wait_scatter(bufi, db)

            for j in range(min(depth, IB)):
                rb, mb, _ = bufs[j % nbuf]
                guarded_wait(j % nbuf)
                pltpu.async_copy(y_hbm.at[sb.at[j]], rb, mb)
            for j in range(IB):
                rc, mc, cc = bufs[j % nbuf]
                pltpu.make_async_copy(y_hbm.at[sb.at[j]], rc, mc).wait()
                if j + depth < IB:
                    bn = (j + depth) % nbuf
                    rn, mn, _ = bufs[bn]
                    if j + depth - nbuf < 0:
                        guarded_wait(bn)
                    else:
                        wait_scatter(bn, db)
                    pltpu.async_copy(y_hbm.at[sb.at[j + depth]], rn, mn)
                pltpu.async_copy(rc, acc.at[db.at[j]], cc, add=True)

        fetch_blk(0, sb0, db0, ms0)

        def body(b2, carry):
            b = 2 * b2
            fetch_blk(b + 1, sb1, db1, ms1)
            wait_blk(b, sb0, db0, ms0)
            process_blk(sb0, db0, b > 0)

            @pl.when(b + 2 < nblk)
            def _():
                fetch_blk(b + 2, sb0, db0, ms0)

            wait_blk(b + 1, sb1, db1, ms1)
            process_blk(sb1, db1, None)
            return carry

        lax.fori_loop(0, nblk // 2, body, 0)
        for bufi in range(nbuf):
            wait_scatter(bufi, db1)
        plsc.subcore_barrier()
        pltpu.sync_copy(acc.at[pl.ds(sid * rpt, rpt)],
                        out_hbm.at[cid, pl.ds(sid * rpt, rpt)])

    return pass_kernel


def _sc_pass_call(y, srcp, dstp, zeros, np_rows, nblk, width):
    return _sc_pass_builder(np_rows, nblk, width)(y, srcp, dstp, zeros)


def _deg_of(d_blk):
    return jnp.max(d_blk[0] + d_blk[1], axis=1, keepdims=True)


def _tc_layer(x, d, w1, b1, w2, b2, w3, b3, n, *, pre_norm, sum_parts):
    din = x.shape[-1]
    dout = w1.shape[1]
    dw = d.shape[-1]
    grid = n // RB
    if sum_parts:
        x_spec = pl.BlockSpec((2, RB, din), lambda i: (0, i, 0))
    else:
        x_spec = pl.BlockSpec((RB, din), lambda i: (i, 0))
    w_spec = pl.BlockSpec((din, dout), lambda i: (0, 0))
    b_spec = pl.BlockSpec((1, dout), lambda i: (0, 0))

    def body(x_ref, d_ref, w1_ref, b1_ref, w2_ref, b2_ref, w3_ref, b3_ref,
             o_ref):
        if sum_parts:
            xb = x_ref[0] + x_ref[1]
        else:
            xb = x_ref[...]
        deg = _deg_of(d_ref[...])
        nrm = lax.rsqrt(jnp.where(deg > 0.0, deg, 1.0))
        if pre_norm:
            xb = xb * nrm
        z1 = jnp.maximum(jnp.dot(xb, w1_ref[...], preferred_element_type=_f32)
                         + b1_ref[...], 0.0)
        z2 = jnp.maximum(jnp.dot(xb, w2_ref[...], preferred_element_type=_f32)
                         + b2_ref[...], 0.0)
        z3 = jnp.maximum(jnp.dot(xb, w3_ref[...], preferred_element_type=_f32)
                         + b3_ref[...], 0.0)
        o_ref[...] = (z1 + z2 + z3) * (nrm * (1.0 / 3.0))

    return pl.pallas_call(
        body,
        grid=(grid,),
        in_specs=[
            x_spec,
            pl.BlockSpec((2, RB, dw), lambda i: (0, i, 0)),
            w_spec, b_spec, w_spec, b_spec, w_spec, b_spec,
        ],
        out_specs=pl.BlockSpec((RB, dout), lambda i: (i, 0)),
        out_shape=jax.ShapeDtypeStruct((n, dout), _f32),
    )(x, d, w1, b1.reshape(1, -1), w2, b2.reshape(1, -1),
      w3, b3.reshape(1, -1))


def _tc_mid(p, d, n):
    width = p.shape[2]
    dw = d.shape[-1]
    grid = n // RB

    def body(p_ref, d_ref, o_ref):
        deg = _deg_of(d_ref[...])
        dinv = 1.0 / jnp.where(deg > 0.0, deg, 1.0)
        o_ref[...] = (p_ref[0] + p_ref[1]) * dinv

    return pl.pallas_call(
        body,
        grid=(grid,),
        in_specs=[
            pl.BlockSpec((2, RB, width), lambda i: (0, i, 0)),
            pl.BlockSpec((2, RB, dw), lambda i: (0, i, 0)),
        ],
        out_specs=pl.BlockSpec((RB, width), lambda i: (i, 0)),
        out_shape=jax.ShapeDtypeStruct((n, width), _f32),
    )(p, d)


def _tc_final(u, d, n, n_cls):
    width = u.shape[2]
    dw = d.shape[-1]
    grid = n // RB

    def body(u_ref, d_ref, o_ref):
        deg = _deg_of(d_ref[...])
        nrm = lax.rsqrt(jnp.where(deg > 0.0, deg, 1.0))
        o = (u_ref[0, :, 0:n_cls] + u_ref[1, :, 0:n_cls]) * nrm
        m = jnp.max(o, axis=1, keepdims=True)
        e = jnp.exp(o - m)
        o_ref[...] = o - m - jnp.log(jnp.sum(e, axis=1, keepdims=True))

    return pl.pallas_call(
        body,
        grid=(grid,),
        in_specs=[
            pl.BlockSpec((2, RB, width), lambda i: (0, i, 0)),
            pl.BlockSpec((2, RB, dw), lambda i: (0, i, 0)),
        ],
        out_specs=pl.BlockSpec((RB, n_cls), lambda i: (i, 0)),
        out_shape=jax.ShapeDtypeStruct((n, n_cls), _f32),
    )(u, d)


def kernel(x, edge_index, W11, b11, W12, b12, W13, b13,
           W21, b21, W22, b22, W23, b23):
    n, d_feat = x.shape
    e = edge_index.shape[1]
    n_cls = W21.shape[1]
    np_rows = (n // (NTILE * 8) + 1) * (NTILE * 8)
    nblk = -(-e // (NW * IB * K))
    nblk += nblk % 2
    ep = NW * IB * K * nblk

    src = edge_index[0].astype(jnp.int32)
    dst = edge_index[1].astype(jnp.int32)
    pad = ep - e
    pad_ids = jnp.arange(pad, dtype=jnp.int32)
    srcp = jnp.concatenate(
        [src, pad_ids % jnp.int32(n)]).reshape(NW, nblk, IB, K)
    dstp = jnp.concatenate(
        [dst, jnp.int32(n) + pad_ids % jnp.int32(np_rows - n)]
    ).reshape(NW, nblk, IB, K)
    ones_mat = jnp.ones((n, d_feat), _f32)
    zh = jnp.zeros((np_rows, d_feat), _f32)
    wpad = d_feat - n_cls
    W21p = jnp.pad(W21, ((0, 0), (0, wpad)))
    W22p = jnp.pad(W22, ((0, 0), (0, wpad)))
    W23p = jnp.pad(W23, ((0, 0), (0, wpad)))
    b21p = jnp.pad(b21, (0, wpad))
    b22p = jnp.pad(b22, (0, wpad))
    b23p = jnp.pad(b23, (0, wpad))

    args = (srcp, dstp, zh, np_rows, nblk, d_feat)
    d = _sc_pass_call(ones_mat, srcp, dstp, zh, np_rows, nblk, d_feat)

    y0 = _tc_layer(x, d, W11, b11, W12, b12, W13, b13, n,
                   pre_norm=False, sum_parts=False)
    p = _sc_pass_call(y0, *args)
    t = _tc_mid(p, d, n)
    q = _sc_pass_call(t, *args)

    y1 = _tc_layer(q, d, W21p, b21p, W22p, b22p, W23p, b23p, n,
                   pre_norm=True, sum_parts=True)
    r = _sc_pass_call(y1, *args)
    t2 = _tc_mid(r, d, n)
    u = _sc_pass_call(t2, *args)

    return _tc_final(u, d, n, n_cls)
